# baseline (speedup 1.0000x reference)
import jax
import jax.numpy as jnp
import numpy as np
from jax import lax
from jax.experimental import pallas as pl
from jax.experimental.pallas import tpu as pltpu
from jax.experimental.pallas import tpu_sc as plsc

N = 10000
E = 320000
NH = 8
HID = 8
NCLS = 41
C = 80
NW = 32
ROWS_PER_TILE = N // 16


def _sc_edge_phase(table, sl_flat, adj, heads, w, f):
    ept = E // NW
    nch = ept // C
    adj4 = adj.reshape(2, 2, E // C, C)
    zeros_rows = jnp.zeros((ROWS_PER_TILE, w), jnp.float32)

    NB = 4

    def body(table_h, sl_h, adj_h, zeros_h, out_h, sl_v, src_a, dst_a,
             rows0, rows1, rows2, rows3, vals0, vals1, vals2, vals3, accum,
             gsem0, gsem1, gsem2, gsem3, vsem0, vsem1, vsem2, vsem3):
        ci = lax.axis_index("c")
        si = lax.axis_index("s")
        wid = si * 2 + ci
        row0 = si * ROWS_PER_TILE
        ch0 = wid * nch
        rows = (rows0, rows1, rows2, rows3)
        vals = (vals0, vals1, vals2, vals3)
        gsem = (gsem0, gsem1, gsem2, gsem3)
        vsem = (vsem0, vsem1, vsem2, vsem3)

        for b in range(NB):
            pltpu.sync_copy(zeros_h.at[pl.ds(0, C)], vals[b])

        for g in range(2):
            pltpu.sync_copy(sl_h.at[g], sl_v)
            pltpu.sync_copy(adj_h.at[g, 0, pl.ds(ch0, nch)], src_a)
            pltpu.sync_copy(adj_h.at[g, 1, pl.ds(ch0, nch)], dst_a)
            pltpu.sync_copy(zeros_h, accum.at[pl.ds(row0, ROWS_PER_TILE)])
            plsc.subcore_barrier()

            for b in range(NB - 1):
                pltpu.async_copy(table_h.at[g].at[dst_a.at[b]], rows[b],
                                 gsem[b])

            @pl.loop(0, (nch + NB - 1) // NB)
            def _chunks(kk):
                for b in range(NB):
                    k = kk * NB + b

                    @pl.when(k < nch)
                    def _():
                        @pl.when(k + NB - 1 < nch)
                        def _():
                            bp = (b + NB - 1) % NB
                            pltpu.async_copy(
                                table_h.at[g].at[dst_a.at[k + NB - 1]],
                                rows[bp], gsem[bp])

                        pltpu.make_async_copy(
                            table_h.at[g].at[dst_a.at[k]], rows[b],
                            gsem[b]).wait()

                        @pl.when(k >= NB)
                        def _():
                            pltpu.make_async_copy(
                                vals[b], accum.at[src_a.at[k]],
                                vsem[b]).wait()

                        @pl.loop(0, C // 16)
                        def _vec(j16):
                            r16 = lax.iota(jnp.int32, 16) + j16 * 16
                            srci = src_a[k, pl.ds(j16 * 16, 16)]
                            slbase = srci * heads
                            for h in range(heads):
                                slh = plsc.load_gather(sl_v, [slbase + h])
                                cvec = jnp.full((16,), f + h, jnp.int32)
                                srh = plsc.load_gather(rows[b], [r16, cvec])
                                e = slh + srh
                                ee = jnp.exp(-jnp.maximum(e, 0.2 * e))
                                plsc.store_scatter(vals[b], [r16, cvec], ee)
                                lo = h * HID if heads > 1 else 0
                                hi = (h + 1) * HID if heads > 1 else f
                                for col in range(lo, hi):
                                    jvec = jnp.full((16,), col, jnp.int32)
                                    hv = plsc.load_gather(rows[b],
                                                          [r16, jvec])
                                    plsc.store_scatter(vals[b], [r16, jvec],
                                                       ee * hv)

                        pltpu.async_copy(vals[b], accum.at[src_a.at[k]],
                                         vsem[b], add=True)

            for i in range(NB):
                k = nch - 1 - i
                pltpu.make_async_copy(vals[k % NB],
                                      accum.at[src_a.at[k]],
                                      vsem[k % NB]).wait()
            plsc.subcore_barrier()
            pltpu.sync_copy(accum.at[pl.ds(row0, ROWS_PER_TILE)],
                            out_h.at[ci, g, pl.ds(row0, ROWS_PER_TILE)])
            plsc.subcore_barrier()

    fn = pl.kernel(
        body,
        out_type=jax.ShapeDtypeStruct((2, 2, N, w), jnp.float32),
        mesh=plsc.VectorSubcoreMesh(core_axis_name="c", subcore_axis_name="s"),
        compiler_params=pltpu.CompilerParams(use_tc_tiling_on_sc=False,
                                             needs_layout_passes=False),
        scratch_types=[
            pltpu.VMEM((N * heads,), jnp.float32),
            pltpu.VMEM((nch, C), jnp.int32),
            pltpu.VMEM((nch, C), jnp.int32),
            pltpu.VMEM((C, w), jnp.float32),
            pltpu.VMEM((C, w), jnp.float32),
            pltpu.VMEM((C, w), jnp.float32),
            pltpu.VMEM((C, w), jnp.float32),
            pltpu.VMEM((C, w), jnp.float32),
            pltpu.VMEM((C, w), jnp.float32),
            pltpu.VMEM((C, w), jnp.float32),
            pltpu.VMEM((C, w), jnp.float32),
            pltpu.VMEM_SHARED((N, w), jnp.float32),
            pltpu.SemaphoreType.DMA,
            pltpu.SemaphoreType.DMA,
            pltpu.SemaphoreType.DMA,
            pltpu.SemaphoreType.DMA,
            pltpu.SemaphoreType.DMA,
            pltpu.SemaphoreType.DMA,
            pltpu.SemaphoreType.DMA,
            pltpu.SemaphoreType.DMA,
        ],
    )
    return fn(table, sl_flat, adj4, zeros_rows)


_BLK = 2000


def _tc_a(x_ref, w_ref, al_ref, ar_ref, taba_ref, tabb_ref, sla_ref, slb_ref):
    x = x_ref[...]
    z4 = jnp.zeros((_BLK, 4), jnp.float32)
    for g in range(2):
        h = jnp.dot(x, w_ref[g], preferred_element_type=jnp.float32)
        sr = jnp.dot(h, ar_ref[g], preferred_element_type=jnp.float32)
        sl = jnp.dot(h, al_ref[g], preferred_element_type=jnp.float32)
        taba_ref[g] = jnp.concatenate([h[:, :32], sr[:, :4], z4], axis=-1)
        tabb_ref[g] = jnp.concatenate([h[:, 32:], sr[:, 4:], z4], axis=-1)
        sla_ref[g] = sl[:, :4]
        slb_ref[g] = sl[:, 4:]


def _elu(x):
    return jnp.where(x > 0, x, jnp.exp(x) - 1.0)


def _tc_b(acca_ref, accb_ref, wout_ref, a2_ref, r8_ref, tab2_ref, sl2_ref):
    z6 = jnp.zeros((_BLK, 48 - NCLS - 1), jnp.float32)
    for g in range(2):
        acca = acca_ref[0, g] + acca_ref[1, g]
        accb = accb_ref[0, g] + accb_ref[1, g]
        hp = jnp.concatenate([acca[:, :32], accb[:, :32]], axis=-1)
        rs = jnp.concatenate([acca[:, 32:36], accb[:, 32:36]], axis=-1)
        denom = jnp.dot(rs, r8_ref[...], preferred_element_type=jnp.float32)
        hcat = _elu(hp / (denom + 1e-16))
        oh = jnp.dot(hcat, wout_ref[g], preferred_element_type=jnp.float32)
        sl2 = jnp.sum(oh * a2_ref[g, :NCLS][None, :], axis=-1)
        sr2 = jnp.sum(oh * a2_ref[g, NCLS:][None, :], axis=-1)
        tab2_ref[g] = jnp.concatenate([oh, sr2[:, None], z6], axis=-1)
        sl2_ref[g] = sl2[:, None]


def _log_softmax(z):
    m = jnp.max(z, axis=-1, keepdims=True)
    s = z - m
    return s - jnp.log(jnp.sum(jnp.exp(s), axis=-1, keepdims=True))


def _tc_c(acc_ref, wqt_ref, bq_ref, wa_ref, wct_ref, bc_ref, out_ref):
    lg = []
    vu = []
    for g in range(2):
        acc = acc_ref[0, g] + acc_ref[1, g]
        hp = acc[:, :NCLS]
        rs = acc[:, NCLS:NCLS + 1]
        o = _elu(hp / (rs + 1e-16))
        l = _log_softmax(o)
        q = jnp.tanh(jnp.dot(l, wqt_ref[...], preferred_element_type=jnp.float32)
                     + bq_ref[...][None, :])
        vu.append(jnp.sum(q * wa_ref[...][0][None, :], axis=-1, keepdims=True))
        lg.append(l)
    m = jnp.maximum(vu[0], vu[1])
    e0 = jnp.exp(vu[0] - m)
    e1 = jnp.exp(vu[1] - m)
    inv = 1.0 / (e0 + e1)
    output = (e0 * inv) * lg[0] + (e1 * inv) * lg[1]
    z = jnp.dot(output, wct_ref[...], preferred_element_type=jnp.float32) \
        + bc_ref[...][None, :]
    out_ref[...] = _log_softmax(z)


def kernel(inputs, adj, gat1_W, gat1_a, gat1_Wout, gat1_aout,
           gat2_W, gat2_a, gat2_Wout, gat2_aout, Wq, bq, Wa, Wc, bc):
    nblk = N // _BLK

    Wcat = jnp.stack([gat1_W, gat2_W]).transpose(0, 2, 1, 3).reshape(2, 128, 64)
    a_stack = jnp.stack([gat1_a, gat2_a])[:, :, 0, :]
    mt = np.zeros((64, NH), np.float32)
    for h in range(NH):
        mt[h * HID:(h + 1) * HID, h] = 1.0
    Al = a_stack[:, :, :HID].reshape(2, 64)[:, :, None] * mt[None]
    Ar = a_stack[:, :, HID:].reshape(2, 64)[:, :, None] * mt[None]
    r8 = np.zeros((NH, 64), np.float32)
    for h in range(NH):
        r8[h, h * HID:(h + 1) * HID] = 1.0
    r8 = jnp.asarray(r8)
    Wout = jnp.stack([gat1_Wout, gat2_Wout])
    aout = jnp.stack([gat1_aout, gat2_aout])[:, 0]

    grid = (nblk,)
    taba, tabb, sla, slb = pl.pallas_call(
        _tc_a,
        grid=grid,
        in_specs=[
            pl.BlockSpec((_BLK, 128), lambda i: (i, 0)),
            pl.BlockSpec((2, 128, 64), lambda i: (0, 0, 0)),
            pl.BlockSpec((2, 64, NH), lambda i: (0, 0, 0)),
            pl.BlockSpec((2, 64, NH), lambda i: (0, 0, 0)),
        ],
        out_specs=[
            pl.BlockSpec((2, _BLK, 40), lambda i: (0, i, 0)),
            pl.BlockSpec((2, _BLK, 40), lambda i: (0, i, 0)),
            pl.BlockSpec((2, _BLK, 4), lambda i: (0, i, 0)),
            pl.BlockSpec((2, _BLK, 4), lambda i: (0, i, 0)),
        ],
        out_shape=[
            jax.ShapeDtypeStruct((2, N, 40), jnp.float32),
            jax.ShapeDtypeStruct((2, N, 40), jnp.float32),
            jax.ShapeDtypeStruct((2, N, 4), jnp.float32),
            jax.ShapeDtypeStruct((2, N, 4), jnp.float32),
        ],
    )(inputs, Wcat, Al, Ar)

    acc1a = _sc_edge_phase(taba, sla.reshape(2, N * 4), adj,
                           heads=4, w=40, f=32)
    acc1b = _sc_edge_phase(tabb, slb.reshape(2, N * 4), adj,
                           heads=4, w=40, f=32)

    table2, sl2 = pl.pallas_call(
        _tc_b,
        grid=grid,
        in_specs=[
            pl.BlockSpec((2, 2, _BLK, 40), lambda i: (0, 0, i, 0)),
            pl.BlockSpec((2, 2, _BLK, 40), lambda i: (0, 0, i, 0)),
            pl.BlockSpec((2, 64, NCLS), lambda i: (0, 0, 0)),
            pl.BlockSpec((2, 2 * NCLS), lambda i: (0, 0)),
            pl.BlockSpec((NH, 64), lambda i: (0, 0)),
        ],
        out_specs=[
            pl.BlockSpec((2, _BLK, 48), lambda i: (0, i, 0)),
            pl.BlockSpec((2, _BLK, 1), lambda i: (0, i, 0)),
        ],
        out_shape=[
            jax.ShapeDtypeStruct((2, N, 48), jnp.float32),
            jax.ShapeDtypeStruct((2, N, 1), jnp.float32),
        ],
    )(acc1a, acc1b, Wout, aout, r8)

    acc2 = _sc_edge_phase(table2, sl2.reshape(2, N), adj, heads=1, w=48, f=NCLS)

    logits = pl.pallas_call(
        _tc_c,
        grid=grid,
        in_specs=[
            pl.BlockSpec((2, 2, _BLK, 48), lambda i: (0, 0, i, 0)),
            pl.BlockSpec((NCLS, 64), lambda i: (0, 0)),
            pl.BlockSpec((64,), lambda i: (0,)),
            pl.BlockSpec((1, 64), lambda i: (0, 0)),
            pl.BlockSpec((NCLS, NCLS), lambda i: (0, 0)),
            pl.BlockSpec((NCLS,), lambda i: (0,)),
        ],
        out_specs=pl.BlockSpec((_BLK, NCLS), lambda i: (i, 0)),
        out_shape=jax.ShapeDtypeStruct((N, NCLS), jnp.float32),
    )(acc2, Wq.T, bq, Wa, Wc.T, bc)

    return (logits, inputs)


# speedup vs baseline: 2.3553x; 2.3553x over previous
import jax
import jax.numpy as jnp
import numpy as np
from jax import lax
from jax.experimental import pallas as pl
from jax.experimental.pallas import tpu as pltpu
from jax.experimental.pallas import tpu_sc as plsc

N = 10000
E = 320000
NH = 8
HID = 8
NCLS = 41
C = 80
NW = 32
ROWS_PER_TILE = N // 16


def _sc_edge_phase(table, sl_flat, adj, heads, w, f):
    ept = E // NW
    nch = ept // C
    adj4 = adj.reshape(2, 2, E // C, C)
    zeros_rows = jnp.zeros((ROWS_PER_TILE, w), jnp.float32)

    NB = 4

    def body(table_h, sl_h, adj_h, zeros_h, out_h, sl_v, src_a, dst_a,
             rows0, rows1, rows2, rows3, vals0, vals1, vals2, vals3, accum,
             gsem0, gsem1, gsem2, gsem3, vsem0, vsem1, vsem2, vsem3):
        ci = lax.axis_index("c")
        si = lax.axis_index("s")
        wid = si * 2 + ci
        row0 = si * ROWS_PER_TILE
        ch0 = wid * nch
        rows = (rows0, rows1, rows2, rows3)
        vals = (vals0, vals1, vals2, vals3)
        gsem = (gsem0, gsem1, gsem2, gsem3)
        vsem = (vsem0, vsem1, vsem2, vsem3)

        for b in range(NB):
            pltpu.sync_copy(zeros_h.at[pl.ds(0, C)], vals[b])

        for g in range(2):
            pltpu.sync_copy(sl_h.at[g], sl_v)
            pltpu.sync_copy(adj_h.at[g, 0, pl.ds(ch0, nch)], src_a)
            pltpu.sync_copy(adj_h.at[g, 1, pl.ds(ch0, nch)], dst_a)
            pltpu.sync_copy(zeros_h, accum.at[pl.ds(row0, ROWS_PER_TILE)])
            plsc.subcore_barrier()

            for b in range(NB - 1):
                pltpu.async_copy(table_h.at[g].at[dst_a.at[b]], rows[b],
                                 gsem[b])

            @pl.loop(0, (nch + NB - 1) // NB)
            def _chunks(kk):
                for b in range(NB):
                    k = kk * NB + b

                    @pl.when(k < nch)
                    def _():
                        @pl.when(k + NB - 1 < nch)
                        def _():
                            bp = (b + NB - 1) % NB
                            pltpu.async_copy(
                                table_h.at[g].at[dst_a.at[k + NB - 1]],
                                rows[bp], gsem[bp])

                        pltpu.make_async_copy(
                            table_h.at[g].at[dst_a.at[k]], rows[b],
                            gsem[b]).wait()

                        @pl.when(k >= NB)
                        def _():
                            pltpu.make_async_copy(
                                vals[b], accum.at[src_a.at[k]],
                                vsem[b]).wait()

                        @pl.loop(0, C // 16)
                        def _vec(j16):
                            r16 = lax.iota(jnp.int32, 16) + j16 * 16
                            srci = src_a[k, pl.ds(j16 * 16, 16)]
                            slbase = srci * heads

                            def cv(col):
                                return jnp.full((16,), col, jnp.int32)

                            def ee_for(h):
                                slh = plsc.load_gather(sl_v, [slbase + h])
                                srh = plsc.load_gather(rows[b],
                                                       [r16, cv(f + h)])
                                e = slh + srh
                                return jnp.exp(-jnp.maximum(e, 0.2 * e))

                            if heads > 1:
                                for h0 in range(0, heads, 2):
                                    ees = (ee_for(h0), ee_for(h0 + 1))
                                    hvs = [
                                        (i, col, plsc.load_gather(
                                            rows[b], [r16, cv(col)]))
                                        for i, h in enumerate((h0, h0 + 1))
                                        for col in range(h * HID,
                                                         (h + 1) * HID)]
                                    plsc.store_scatter(
                                        vals[b], [r16, cv(f + h0)], ees[0])
                                    plsc.store_scatter(
                                        vals[b], [r16, cv(f + h0 + 1)],
                                        ees[1])
                                    for i, col, hv in hvs:
                                        plsc.store_scatter(
                                            vals[b], [r16, cv(col)],
                                            ees[i] * hv)
                            else:
                                ee = ee_for(0)
                                plsc.store_scatter(vals[b], [r16, cv(f)], ee)
                                for c0 in range(0, f, 14):
                                    cs = range(c0, min(c0 + 14, f))
                                    hvs = [(col, plsc.load_gather(
                                        rows[b], [r16, cv(col)]))
                                        for col in cs]
                                    for col, hv in hvs:
                                        plsc.store_scatter(
                                            vals[b], [r16, cv(col)],
                                            ee * hv)

                        pltpu.async_copy(vals[b], accum.at[src_a.at[k]],
                                         vsem[b], add=True)

            for i in range(NB):
                k = nch - 1 - i
                pltpu.make_async_copy(vals[k % NB],
                                      accum.at[src_a.at[k]],
                                      vsem[k % NB]).wait()
            plsc.subcore_barrier()
            pltpu.sync_copy(accum.at[pl.ds(row0, ROWS_PER_TILE)],
                            out_h.at[ci, g, pl.ds(row0, ROWS_PER_TILE)])
            plsc.subcore_barrier()

    fn = pl.kernel(
        body,
        out_type=jax.ShapeDtypeStruct((2, 2, N, w), jnp.float32),
        mesh=plsc.VectorSubcoreMesh(core_axis_name="c", subcore_axis_name="s"),
        compiler_params=pltpu.CompilerParams(use_tc_tiling_on_sc=False,
                                             needs_layout_passes=False),
        scratch_types=[
            pltpu.VMEM((N * heads,), jnp.float32),
            pltpu.VMEM((nch, C), jnp.int32),
            pltpu.VMEM((nch, C), jnp.int32),
            pltpu.VMEM((C, w), jnp.float32),
            pltpu.VMEM((C, w), jnp.float32),
            pltpu.VMEM((C, w), jnp.float32),
            pltpu.VMEM((C, w), jnp.float32),
            pltpu.VMEM((C, w), jnp.float32),
            pltpu.VMEM((C, w), jnp.float32),
            pltpu.VMEM((C, w), jnp.float32),
            pltpu.VMEM((C, w), jnp.float32),
            pltpu.VMEM_SHARED((N, w), jnp.float32),
            pltpu.SemaphoreType.DMA,
            pltpu.SemaphoreType.DMA,
            pltpu.SemaphoreType.DMA,
            pltpu.SemaphoreType.DMA,
            pltpu.SemaphoreType.DMA,
            pltpu.SemaphoreType.DMA,
            pltpu.SemaphoreType.DMA,
            pltpu.SemaphoreType.DMA,
        ],
    )
    return fn(table, sl_flat, adj4, zeros_rows)


_BLK = 2000


def _tc_a(x_ref, w_ref, al_ref, ar_ref, taba_ref, tabb_ref, sla_ref, slb_ref):
    x = x_ref[...]
    z4 = jnp.zeros((_BLK, 4), jnp.float32)
    for g in range(2):
        h = jnp.dot(x, w_ref[g], preferred_element_type=jnp.float32)
        sr = jnp.dot(h, ar_ref[g], preferred_element_type=jnp.float32)
        sl = jnp.dot(h, al_ref[g], preferred_element_type=jnp.float32)
        taba_ref[g] = jnp.concatenate([h[:, :32], sr[:, :4], z4], axis=-1)
        tabb_ref[g] = jnp.concatenate([h[:, 32:], sr[:, 4:], z4], axis=-1)
        sla_ref[g] = sl[:, :4]
        slb_ref[g] = sl[:, 4:]


def _elu(x):
    return jnp.where(x > 0, x, jnp.exp(x) - 1.0)


def _tc_b(acca_ref, accb_ref, wout_ref, a2_ref, r8_ref, tab2_ref, sl2_ref):
    z6 = jnp.zeros((_BLK, 48 - NCLS - 1), jnp.float32)
    for g in range(2):
        acca = acca_ref[0, g] + acca_ref[1, g]
        accb = accb_ref[0, g] + accb_ref[1, g]
        hp = jnp.concatenate([acca[:, :32], accb[:, :32]], axis=-1)
        rs = jnp.concatenate([acca[:, 32:36], accb[:, 32:36]], axis=-1)
        denom = jnp.dot(rs, r8_ref[...], preferred_element_type=jnp.float32)
        hcat = _elu(hp / (denom + 1e-16))
        oh = jnp.dot(hcat, wout_ref[g], preferred_element_type=jnp.float32)
        sl2 = jnp.sum(oh * a2_ref[g, :NCLS][None, :], axis=-1)
        sr2 = jnp.sum(oh * a2_ref[g, NCLS:][None, :], axis=-1)
        tab2_ref[g] = jnp.concatenate([oh, sr2[:, None], z6], axis=-1)
        sl2_ref[g] = sl2[:, None]


def _log_softmax(z):
    m = jnp.max(z, axis=-1, keepdims=True)
    s = z - m
    return s - jnp.log(jnp.sum(jnp.exp(s), axis=-1, keepdims=True))


def _tc_c(acc_ref, wqt_ref, bq_ref, wa_ref, wct_ref, bc_ref, out_ref):
    lg = []
    vu = []
    for g in range(2):
        acc = acc_ref[0, g] + acc_ref[1, g]
        hp = acc[:, :NCLS]
        rs = acc[:, NCLS:NCLS + 1]
        o = _elu(hp / (rs + 1e-16))
        l = _log_softmax(o)
        q = jnp.tanh(jnp.dot(l, wqt_ref[...], preferred_element_type=jnp.float32)
                     + bq_ref[...][None, :])
        vu.append(jnp.sum(q * wa_ref[...][0][None, :], axis=-1, keepdims=True))
        lg.append(l)
    m = jnp.maximum(vu[0], vu[1])
    e0 = jnp.exp(vu[0] - m)
    e1 = jnp.exp(vu[1] - m)
    inv = 1.0 / (e0 + e1)
    output = (e0 * inv) * lg[0] + (e1 * inv) * lg[1]
    z = jnp.dot(output, wct_ref[...], preferred_element_type=jnp.float32) \
        + bc_ref[...][None, :]
    out_ref[...] = _log_softmax(z)


def kernel(inputs, adj, gat1_W, gat1_a, gat1_Wout, gat1_aout,
           gat2_W, gat2_a, gat2_Wout, gat2_aout, Wq, bq, Wa, Wc, bc):
    nblk = N // _BLK

    Wcat = jnp.stack([gat1_W, gat2_W]).transpose(0, 2, 1, 3).reshape(2, 128, 64)
    a_stack = jnp.stack([gat1_a, gat2_a])[:, :, 0, :]
    mt = np.zeros((64, NH), np.float32)
    for h in range(NH):
        mt[h * HID:(h + 1) * HID, h] = 1.0
    Al = a_stack[:, :, :HID].reshape(2, 64)[:, :, None] * mt[None]
    Ar = a_stack[:, :, HID:].reshape(2, 64)[:, :, None] * mt[None]
    r8 = np.zeros((NH, 64), np.float32)
    for h in range(NH):
        r8[h, h * HID:(h + 1) * HID] = 1.0
    r8 = jnp.asarray(r8)
    Wout = jnp.stack([gat1_Wout, gat2_Wout])
    aout = jnp.stack([gat1_aout, gat2_aout])[:, 0]

    grid = (nblk,)
    taba, tabb, sla, slb = pl.pallas_call(
        _tc_a,
        grid=grid,
        in_specs=[
            pl.BlockSpec((_BLK, 128), lambda i: (i, 0)),
            pl.BlockSpec((2, 128, 64), lambda i: (0, 0, 0)),
            pl.BlockSpec((2, 64, NH), lambda i: (0, 0, 0)),
            pl.BlockSpec((2, 64, NH), lambda i: (0, 0, 0)),
        ],
        out_specs=[
            pl.BlockSpec((2, _BLK, 40), lambda i: (0, i, 0)),
            pl.BlockSpec((2, _BLK, 40), lambda i: (0, i, 0)),
            pl.BlockSpec((2, _BLK, 4), lambda i: (0, i, 0)),
            pl.BlockSpec((2, _BLK, 4), lambda i: (0, i, 0)),
        ],
        out_shape=[
            jax.ShapeDtypeStruct((2, N, 40), jnp.float32),
            jax.ShapeDtypeStruct((2, N, 40), jnp.float32),
            jax.ShapeDtypeStruct((2, N, 4), jnp.float32),
            jax.ShapeDtypeStruct((2, N, 4), jnp.float32),
        ],
    )(inputs, Wcat, Al, Ar)

    acc1a = _sc_edge_phase(taba, sla.reshape(2, N * 4), adj,
                           heads=4, w=40, f=32)
    acc1b = _sc_edge_phase(tabb, slb.reshape(2, N * 4), adj,
                           heads=4, w=40, f=32)

    table2, sl2 = pl.pallas_call(
        _tc_b,
        grid=grid,
        in_specs=[
            pl.BlockSpec((2, 2, _BLK, 40), lambda i: (0, 0, i, 0)),
            pl.BlockSpec((2, 2, _BLK, 40), lambda i: (0, 0, i, 0)),
            pl.BlockSpec((2, 64, NCLS), lambda i: (0, 0, 0)),
            pl.BlockSpec((2, 2 * NCLS), lambda i: (0, 0)),
            pl.BlockSpec((NH, 64), lambda i: (0, 0)),
        ],
        out_specs=[
            pl.BlockSpec((2, _BLK, 48), lambda i: (0, i, 0)),
            pl.BlockSpec((2, _BLK, 1), lambda i: (0, i, 0)),
        ],
        out_shape=[
            jax.ShapeDtypeStruct((2, N, 48), jnp.float32),
            jax.ShapeDtypeStruct((2, N, 1), jnp.float32),
        ],
    )(acc1a, acc1b, Wout, aout, r8)

    acc2 = _sc_edge_phase(table2, sl2.reshape(2, N), adj, heads=1, w=48, f=NCLS)

    logits = pl.pallas_call(
        _tc_c,
        grid=grid,
        in_specs=[
            pl.BlockSpec((2, 2, _BLK, 48), lambda i: (0, 0, i, 0)),
            pl.BlockSpec((NCLS, 64), lambda i: (0, 0)),
            pl.BlockSpec((64,), lambda i: (0,)),
            pl.BlockSpec((1, 64), lambda i: (0, 0)),
            pl.BlockSpec((NCLS, NCLS), lambda i: (0, 0)),
            pl.BlockSpec((NCLS,), lambda i: (0,)),
        ],
        out_specs=pl.BlockSpec((_BLK, NCLS), lambda i: (i, 0)),
        out_shape=jax.ShapeDtypeStruct((N, NCLS), jnp.float32),
    )(acc2, Wq.T, bq, Wa, Wc.T, bc)

    return (logits, inputs)


# speedup vs baseline: 2.3588x; 1.0015x over previous
import jax
import jax.numpy as jnp
import numpy as np
from jax import lax
from jax.experimental import pallas as pl
from jax.experimental.pallas import tpu as pltpu
from jax.experimental.pallas import tpu_sc as plsc

N = 10000
E = 320000
NH = 8
HID = 8
NCLS = 41
C = 80
NW = 32
ROWS_PER_TILE = N // 16
NB = 4


def _sc_edge_phase(table, sl_flat, adj4, heads, w, f):
    ept = E // NW
    nch = ept // C
    zeros_rows = jnp.zeros((ROWS_PER_TILE, w), jnp.float32)

    def body(table_h, sl_h, adj_h, zeros_h, out_h, sl_v, src_a, dst_a,
             rows0, rows1, rows2, rows3, vals0, vals1, vals2, vals3, accum,
             gsem0, gsem1, gsem2, gsem3, vsem0, vsem1, vsem2, vsem3):
        ci = lax.axis_index("c")
        si = lax.axis_index("s")
        wid = si * 2 + ci
        row0 = si * ROWS_PER_TILE
        ch0 = wid * nch
        rows = (rows0, rows1, rows2, rows3)
        vals = (vals0, vals1, vals2, vals3)
        gsem = (gsem0, gsem1, gsem2, gsem3)
        vsem = (vsem0, vsem1, vsem2, vsem3)

        for b in range(NB):
            pltpu.sync_copy(zeros_h.at[pl.ds(0, C)], vals[b])

        for g in range(2):
            pltpu.sync_copy(sl_h.at[g], sl_v)
            pltpu.sync_copy(adj_h.at[g, 0, pl.ds(ch0, nch)], src_a)
            pltpu.sync_copy(adj_h.at[g, 1, pl.ds(ch0, nch)], dst_a)
            pltpu.sync_copy(zeros_h, accum.at[pl.ds(row0, ROWS_PER_TILE)])
            plsc.subcore_barrier()

            for b in range(NB - 1):
                pltpu.async_copy(table_h.at[g].at[dst_a.at[b]], rows[b],
                                 gsem[b])

            @pl.loop(0, (nch + NB - 1) // NB)
            def _chunks(kk):
                for b in range(NB):
                    k = kk * NB + b

                    @pl.when(k < nch)
                    def _():
                        @pl.when(k + NB - 1 < nch)
                        def _():
                            bp = (b + NB - 1) % NB
                            pltpu.async_copy(
                                table_h.at[g].at[dst_a.at[k + NB - 1]],
                                rows[bp], gsem[bp])

                        pltpu.make_async_copy(
                            table_h.at[g].at[dst_a.at[k]], rows[b],
                            gsem[b]).wait()

                        @pl.when(k >= NB)
                        def _():
                            pltpu.make_async_copy(
                                vals[b], accum.at[src_a.at[k]],
                                vsem[b]).wait()

                        @pl.loop(0, C // 16)
                        def _vec(j16):
                            r16 = lax.iota(jnp.int32, 16) + j16 * 16
                            srci = src_a[k, pl.ds(j16 * 16, 16)]
                            slbase = srci * heads

                            def cv(col):
                                return jnp.full((16,), col, jnp.int32)

                            def ee_for(h):
                                slh = plsc.load_gather(sl_v, [slbase + h])
                                srh = plsc.load_gather(rows[b],
                                                       [r16, cv(f + h)])
                                e = slh + srh
                                return jnp.exp(-jnp.maximum(e, 0.2 * e))

                            if heads > 1:
                                for h0 in range(0, heads, 2):
                                    ees = (ee_for(h0), ee_for(h0 + 1))
                                    hvs = [
                                        (i, col, plsc.load_gather(
                                            rows[b], [r16, cv(col)]))
                                        for i, h in enumerate((h0, h0 + 1))
                                        for col in range(h * HID,
                                                         (h + 1) * HID)]
                                    plsc.store_scatter(
                                        vals[b], [r16, cv(f + h0)], ees[0])
                                    plsc.store_scatter(
                                        vals[b], [r16, cv(f + h0 + 1)],
                                        ees[1])
                                    for i, col, hv in hvs:
                                        plsc.store_scatter(
                                            vals[b], [r16, cv(col)],
                                            ees[i] * hv)
                            else:
                                ee = ee_for(0)
                                plsc.store_scatter(vals[b], [r16, cv(f)], ee)
                                for c0 in range(0, f, 14):
                                    cs = range(c0, min(c0 + 14, f))
                                    hvs = [(col, plsc.load_gather(
                                        rows[b], [r16, cv(col)]))
                                        for col in cs]
                                    for col, hv in hvs:
                                        plsc.store_scatter(
                                            vals[b], [r16, cv(col)],
                                            ee * hv)

                        pltpu.async_copy(vals[b], accum.at[src_a.at[k]],
                                         vsem[b], add=True)

            for i in range(NB):
                k = nch - 1 - i
                pltpu.make_async_copy(vals[k % NB],
                                      accum.at[src_a.at[k]],
                                      vsem[k % NB]).wait()
            plsc.subcore_barrier()
            pltpu.sync_copy(accum.at[pl.ds(row0, ROWS_PER_TILE)],
                            out_h.at[ci, g, pl.ds(row0, ROWS_PER_TILE)])
            plsc.subcore_barrier()

    fn = pl.kernel(
        body,
        out_type=jax.ShapeDtypeStruct((2, 2, N, w), jnp.float32),
        mesh=plsc.VectorSubcoreMesh(core_axis_name="c", subcore_axis_name="s"),
        compiler_params=pltpu.CompilerParams(use_tc_tiling_on_sc=False,
                                             needs_layout_passes=False),
        scratch_types=[
            pltpu.VMEM((N * heads,), jnp.float32),
            pltpu.VMEM((E // NW // C, C), jnp.int32),
            pltpu.VMEM((E // NW // C, C), jnp.int32),
            pltpu.VMEM((C, w), jnp.float32),
            pltpu.VMEM((C, w), jnp.float32),
            pltpu.VMEM((C, w), jnp.float32),
            pltpu.VMEM((C, w), jnp.float32),
            pltpu.VMEM((C, w), jnp.float32),
            pltpu.VMEM((C, w), jnp.float32),
            pltpu.VMEM((C, w), jnp.float32),
            pltpu.VMEM((C, w), jnp.float32),
            pltpu.VMEM_SHARED((N, w), jnp.float32),
            pltpu.SemaphoreType.DMA,
            pltpu.SemaphoreType.DMA,
            pltpu.SemaphoreType.DMA,
            pltpu.SemaphoreType.DMA,
            pltpu.SemaphoreType.DMA,
            pltpu.SemaphoreType.DMA,
            pltpu.SemaphoreType.DMA,
            pltpu.SemaphoreType.DMA,
        ],
    )
    return fn(table, sl_flat, adj4, zeros_rows)


_BLK = 2000


def _tc_a(x_ref, w_ref, al_ref, ar_ref, taba_ref, tabb_ref, sla_ref, slb_ref):
    x = x_ref[...]
    z4 = jnp.zeros((_BLK, 4), jnp.float32)
    for g in range(2):
        h = jnp.dot(x, w_ref[g], preferred_element_type=jnp.float32)
        sr = jnp.dot(h, ar_ref[g], preferred_element_type=jnp.float32)
        sl = jnp.dot(h, al_ref[g], preferred_element_type=jnp.float32)
        taba_ref[g] = jnp.concatenate([h[:, :32], sr[:, :4], z4], axis=-1)
        tabb_ref[g] = jnp.concatenate([h[:, 32:], sr[:, 4:], z4], axis=-1)
        sla_ref[g] = sl[:, :4]
        slb_ref[g] = sl[:, 4:]


def _elu(x):
    return jnp.where(x > 0, x, jnp.exp(x) - 1.0)


def _tc_b(acca_ref, accb_ref, wout_ref, a2_ref, r8_ref, tab2_ref, sl2_ref):
    z6 = jnp.zeros((_BLK, 48 - NCLS - 1), jnp.float32)
    for g in range(2):
        acca = acca_ref[0, g] + acca_ref[1, g]
        accb = accb_ref[0, g] + accb_ref[1, g]
        hp = jnp.concatenate([acca[:, :32], accb[:, :32]], axis=-1)
        rs = jnp.concatenate([acca[:, 32:36], accb[:, 32:36]], axis=-1)
        denom = jnp.dot(rs, r8_ref[...], preferred_element_type=jnp.float32)
        hcat = _elu(hp / (denom + 1e-16))
        oh = jnp.dot(hcat, wout_ref[g], preferred_element_type=jnp.float32)
        sl2 = jnp.sum(oh * a2_ref[g, :NCLS][None, :], axis=-1)
        sr2 = jnp.sum(oh * a2_ref[g, NCLS:][None, :], axis=-1)
        tab2_ref[g] = jnp.concatenate([oh, sr2[:, None], z6], axis=-1)
        sl2_ref[g] = sl2[:, None]


def _log_softmax(z):
    m = jnp.max(z, axis=-1, keepdims=True)
    s = z - m
    return s - jnp.log(jnp.sum(jnp.exp(s), axis=-1, keepdims=True))


def _tc_c(acc_ref, wqt_ref, bq_ref, wa_ref, wct_ref, bc_ref, out_ref):
    lg = []
    vu = []
    for g in range(2):
        acc = acc_ref[0, g] + acc_ref[1, g]
        hp = acc[:, :NCLS]
        rs = acc[:, NCLS:NCLS + 1]
        o = _elu(hp / (rs + 1e-16))
        l = _log_softmax(o)
        q = jnp.tanh(jnp.dot(l, wqt_ref[...], preferred_element_type=jnp.float32)
                     + bq_ref[...][None, :])
        vu.append(jnp.sum(q * wa_ref[...][0][None, :], axis=-1, keepdims=True))
        lg.append(l)
    m = jnp.maximum(vu[0], vu[1])
    e0 = jnp.exp(vu[0] - m)
    e1 = jnp.exp(vu[1] - m)
    inv = 1.0 / (e0 + e1)
    output = (e0 * inv) * lg[0] + (e1 * inv) * lg[1]
    z = jnp.dot(output, wct_ref[...], preferred_element_type=jnp.float32) \
        + bc_ref[...][None, :]
    out_ref[...] = _log_softmax(z)


def kernel(inputs, adj, gat1_W, gat1_a, gat1_Wout, gat1_aout,
           gat2_W, gat2_a, gat2_Wout, gat2_aout, Wq, bq, Wa, Wc, bc):
    nblk = N // _BLK
    adj4 = adj.reshape(2, 2, E // C, C)

    Wcat = jnp.stack([gat1_W, gat2_W]).transpose(0, 2, 1, 3).reshape(2, 128, 64)
    a_stack = jnp.stack([gat1_a, gat2_a])[:, :, 0, :]
    mt = np.zeros((64, NH), np.float32)
    for h in range(NH):
        mt[h * HID:(h + 1) * HID, h] = 1.0
    Al = a_stack[:, :, :HID].reshape(2, 64)[:, :, None] * mt[None]
    Ar = a_stack[:, :, HID:].reshape(2, 64)[:, :, None] * mt[None]
    r8 = np.zeros((NH, 64), np.float32)
    for h in range(NH):
        r8[h, h * HID:(h + 1) * HID] = 1.0
    r8 = jnp.asarray(r8)
    Wout = jnp.stack([gat1_Wout, gat2_Wout])
    aout = jnp.stack([gat1_aout, gat2_aout])[:, 0]

    grid = (nblk,)
    taba, tabb, sla, slb = pl.pallas_call(
        _tc_a,
        grid=grid,
        in_specs=[
            pl.BlockSpec((_BLK, 128), lambda i: (i, 0)),
            pl.BlockSpec((2, 128, 64), lambda i: (0, 0, 0)),
            pl.BlockSpec((2, 64, NH), lambda i: (0, 0, 0)),
            pl.BlockSpec((2, 64, NH), lambda i: (0, 0, 0)),
        ],
        out_specs=[
            pl.BlockSpec((2, _BLK, 40), lambda i: (0, i, 0)),
            pl.BlockSpec((2, _BLK, 40), lambda i: (0, i, 0)),
            pl.BlockSpec((2, _BLK, 4), lambda i: (0, i, 0)),
            pl.BlockSpec((2, _BLK, 4), lambda i: (0, i, 0)),
        ],
        out_shape=[
            jax.ShapeDtypeStruct((2, N, 40), jnp.float32),
            jax.ShapeDtypeStruct((2, N, 40), jnp.float32),
            jax.ShapeDtypeStruct((2, N, 4), jnp.float32),
            jax.ShapeDtypeStruct((2, N, 4), jnp.float32),
        ],
    )(inputs, Wcat, Al, Ar)

    acc1a = _sc_edge_phase(taba, sla.reshape(2, N * 4), adj4,
                           heads=4, w=40, f=32)
    acc1b = _sc_edge_phase(tabb, slb.reshape(2, N * 4), adj4,
                           heads=4, w=40, f=32)

    table2, sl2 = pl.pallas_call(
        _tc_b,
        grid=grid,
        in_specs=[
            pl.BlockSpec((2, 2, _BLK, 40), lambda i: (0, 0, i, 0)),
            pl.BlockSpec((2, 2, _BLK, 40), lambda i: (0, 0, i, 0)),
            pl.BlockSpec((2, 64, NCLS), lambda i: (0, 0, 0)),
            pl.BlockSpec((2, 2 * NCLS), lambda i: (0, 0)),
            pl.BlockSpec((NH, 64), lambda i: (0, 0)),
        ],
        out_specs=[
            pl.BlockSpec((2, _BLK, 48), lambda i: (0, i, 0)),
            pl.BlockSpec((2, _BLK, 1), lambda i: (0, i, 0)),
        ],
        out_shape=[
            jax.ShapeDtypeStruct((2, N, 48), jnp.float32),
            jax.ShapeDtypeStruct((2, N, 1), jnp.float32),
        ],
    )(acc1a, acc1b, Wout, aout, r8)

    acc2 = _sc_edge_phase(table2, sl2.reshape(2, N), adj4,
                          heads=1, w=48, f=NCLS)

    logits = pl.pallas_call(
        _tc_c,
        grid=grid,
        in_specs=[
            pl.BlockSpec((2, 2, _BLK, 48), lambda i: (0, 0, i, 0)),
            pl.BlockSpec((NCLS, 64), lambda i: (0, 0)),
            pl.BlockSpec((64,), lambda i: (0,)),
            pl.BlockSpec((1, 64), lambda i: (0, 0)),
            pl.BlockSpec((NCLS, NCLS), lambda i: (0, 0)),
            pl.BlockSpec((NCLS,), lambda i: (0,)),
        ],
        out_specs=pl.BlockSpec((_BLK, NCLS), lambda i: (i, 0)),
        out_shape=jax.ShapeDtypeStruct((N, NCLS), jnp.float32),
    )(acc2, Wq.T, bq, Wa, Wc.T, bc)

    return (logits, inputs)
